# baseline (speedup 1.0000x reference)
import functools

import jax
import jax.numpy as jnp
from jax import lax
from jax.experimental import pallas as pl
from jax.experimental.pallas import tpu as pltpu
from jax.experimental.pallas import tpu_sc as plsc

N, E, D, H = 10000, 320000, 128, 256


_CHUNK = 80
_NCHUNKS = E // _CHUNK
_NW = 32
_JMAX = (_NCHUNKS + _NW - 1) // _NW
_RPT = 624
_TAIL = N - 16 * _RPT


def _make_sc_aggregate():
    mesh = plsc.VectorSubcoreMesh(core_axis_name="c", subcore_axis_name="s")

    @functools.partial(
        pl.kernel,
        mesh=mesh,
        out_type=jax.ShapeDtypeStruct((2 * N, D), jnp.float32),
        scratch_types=[
            pltpu.VMEM_SHARED((N, D), jnp.float32),
            pltpu.VMEM((_CHUNK, D), jnp.float32),
            pltpu.VMEM((_CHUNK, D), jnp.float32),
            pltpu.VMEM((_CHUNK, D), jnp.float32),
            pltpu.VMEM((_CHUNK, D), jnp.float32),
        ] + [pltpu.VMEM((_CHUNK,), jnp.int32)] * 8
          + [pltpu.SemaphoreType.DMA] * 12,
    )
    def sc_aggregate(x_hbm, src_hbm, dst_hbm, ea_hbm, out_hbm,
                     accum, ea_v, g_v, ea_w, g_w,
                     ix0, jx0, ix1, jx1, ix2, jx2, ix3, jx3,
                     sem_e, sem_g, sem_e2, sem_g2,
                     si0, sj0, si1, sj1, si2, sj2, si3, sj3):
        cid = lax.axis_index("c")
        sid = lax.axis_index("s")
        wid = sid * 2 + cid

        zvec = jnp.zeros((16,), jnp.float32)

        def _zrow(r, carry):
            for k in range(D // 16):
                ea_v[r, pl.ds(k * 16, 16)] = zvec
            return carry

        lax.fori_loop(0, _CHUNK, _zrow, 0)
        zbase = sid * _RPT
        for b in range(_RPT // _CHUNK):
            pltpu.sync_copy(ea_v, accum.at[pl.ds(zbase + b * _CHUNK, _CHUNK)])
        rem = _RPT % _CHUNK
        if rem:
            pltpu.sync_copy(
                ea_v.at[pl.ds(0, rem)],
                accum.at[pl.ds(zbase + (_RPT // _CHUNK) * _CHUNK, rem)])

        @pl.when(sid == 15)
        def _():
            pltpu.sync_copy(ea_v.at[pl.ds(0, _TAIL)],
                            accum.at[pl.ds(16 * _RPT, _TAIL)])

        plsc.subcore_barrier()

        ipairs = [(ix0, jx0, si0, sj0), (ix1, jx1, si1, sj1),
                  (ix2, jx2, si2, sj2), (ix3, jx3, si3, sj3)]
        dpairs = [(ea_v, g_v, sem_e, sem_g), (ea_w, g_w, sem_e2, sem_g2)]

        def _ia(p, j):
            sv, dv, si, sj = p
            chunk = j * _NW + wid

            @pl.when(chunk < _NCHUNKS)
            def _():
                base = chunk * _CHUNK
                pltpu.async_copy(src_hbm.at[pl.ds(base, _CHUNK)], sv, si)
                pltpu.async_copy(dst_hbm.at[pl.ds(base, _CHUNK)], dv, sj)

        def _d(p, b, j):
            sv, dv, si, sj = p
            ev, gv, se, sg = b
            chunk = j * _NW + wid

            @pl.when(chunk < _NCHUNKS)
            def _():
                base = chunk * _CHUNK
                pltpu.async_copy(ea_hbm.at[pl.ds(base, _CHUNK)], ev, se)
                pltpu.make_async_copy(
                    src_hbm.at[pl.ds(0, _CHUNK)], sv, si).wait()
                pltpu.async_copy(x_hbm.at[sv], gv, sg)

        def _f(p, b, j):
            sv, dv, si, sj = p
            ev, gv, se, sg = b
            chunk = j * _NW + wid

            @pl.when(chunk < _NCHUNKS)
            def _():
                pltpu.make_async_copy(
                    dst_hbm.at[pl.ds(0, _CHUNK)], dv, sj).wait()
                pltpu.make_async_copy(
                    ea_hbm.at[pl.ds(0, _CHUNK)], ev, se).wait()
                pltpu.make_async_copy(
                    x_hbm.at[pl.ds(0, _CHUNK)], gv, sg).wait()
                pltpu.async_copy(ev, accum.at[dv], se, add=True)
                pltpu.sync_copy(gv, accum.at[dv], add=True)
                pltpu.make_async_copy(
                    ea_hbm.at[pl.ds(0, _CHUNK)], ev, se).wait()

        for q in range(4):
            _ia(ipairs[q], q)
        _d(ipairs[0], dpairs[0], 0)
        _d(ipairs[1], dpairs[1], 1)

        def _quad_body(t, carry):
            j0 = t * 4
            for u in range(4):
                _f(ipairs[u], dpairs[u % 2], j0 + u)
                _ia(ipairs[u], j0 + u + 4)
                _d(ipairs[(u + 2) % 4], dpairs[u % 2], j0 + u + 2)
            return carry

        lax.fori_loop(0, (_JMAX + 3) // 4, _quad_body, 0)
        plsc.subcore_barrier()

        obase = cid * N + sid * _RPT
        pltpu.sync_copy(accum.at[pl.ds(sid * _RPT, _RPT)],
                        out_hbm.at[pl.ds(obase, _RPT)])

        @pl.when(sid == 15)
        def _():
            pltpu.sync_copy(accum.at[pl.ds(16 * _RPT, _TAIL)],
                            out_hbm.at[pl.ds(cid * N + 16 * _RPT, _TAIL)])

    return sc_aggregate


_sc_aggregate = _make_sc_aggregate()


_BLK = 1000


def _mlp_body(p_ref, x_ref, w1_ref, b1_ref, w2_ref, b2_ref, eps_ref, o_ref):
    aggr = p_ref[0] + p_ref[1]
    h = jnp.dot(aggr, w1_ref[...], preferred_element_type=jnp.float32)
    h = jnp.maximum(h + b1_ref[...], 0.0)
    o = jnp.dot(h, w2_ref[...], preferred_element_type=jnp.float32)
    o_ref[...] = o + b2_ref[...] + (1.0 + eps_ref[0, 0]) * x_ref[...]


def _mlp(partial2, x, W1, b1, W2, b2, eps):
    grid = (N // _BLK,)
    return pl.pallas_call(
        _mlp_body,
        grid=grid,
        in_specs=[
            pl.BlockSpec((2, _BLK, D), lambda i: (0, i, 0)),
            pl.BlockSpec((_BLK, D), lambda i: (i, 0)),
            pl.BlockSpec((D, H), lambda i: (0, 0)),
            pl.BlockSpec((1, H), lambda i: (0, 0)),
            pl.BlockSpec((H, D), lambda i: (0, 0)),
            pl.BlockSpec((1, D), lambda i: (0, 0)),
            pl.BlockSpec((1, 1), lambda i: (0, 0)),
        ],
        out_specs=pl.BlockSpec((_BLK, D), lambda i: (i, 0)),
        out_shape=jax.ShapeDtypeStruct((N, D), jnp.float32),
    )(partial2, x, W1, b1, W2, b2, eps)


def kernel(x, edge_index, edge_attr, W1, b1, W2, b2, eps):
    src = edge_index[0]
    dst = edge_index[1]
    partial = _sc_aggregate(x, src, dst, edge_attr)
    partial2 = partial.reshape(2, N, D)
    return _mlp(partial2, x, W1, b1.reshape(1, H), W2, b2.reshape(1, D),
                eps.reshape(1, 1))


# speedup vs baseline: 1.0844x; 1.0844x over previous
import functools

import jax
import jax.numpy as jnp
from jax import lax
from jax.experimental import pallas as pl
from jax.experimental.pallas import tpu as pltpu
from jax.experimental.pallas import tpu_sc as plsc

N, E, D, H = 10000, 320000, 128, 256


_CHUNK = 80
_NCHUNKS = E // _CHUNK
_NW = 32
_JMAX = (_NCHUNKS + _NW - 1) // _NW
_RPT = 624
_TAIL = N - 16 * _RPT


def _make_sc_aggregate():
    mesh = plsc.VectorSubcoreMesh(core_axis_name="c", subcore_axis_name="s")

    @functools.partial(
        pl.kernel,
        mesh=mesh,
        out_type=jax.ShapeDtypeStruct((2 * N, D), jnp.float32),
        scratch_types=[
            pltpu.VMEM_SHARED((N, D), jnp.float32),
            pltpu.VMEM((_CHUNK,), jnp.int32),
            pltpu.VMEM((_CHUNK,), jnp.int32),
            pltpu.VMEM((_CHUNK, D), jnp.float32),
            pltpu.VMEM((_CHUNK, D), jnp.float32),
            pltpu.VMEM((_CHUNK,), jnp.int32),
            pltpu.VMEM((_CHUNK,), jnp.int32),
            pltpu.VMEM((_CHUNK, D), jnp.float32),
            pltpu.VMEM((_CHUNK, D), jnp.float32),
            pltpu.SemaphoreType.DMA,
            pltpu.SemaphoreType.DMA,
            pltpu.SemaphoreType.DMA,
            pltpu.SemaphoreType.DMA,
            pltpu.SemaphoreType.DMA,
            pltpu.SemaphoreType.DMA,
            pltpu.SemaphoreType.DMA,
            pltpu.SemaphoreType.DMA,
        ],
    )
    def sc_aggregate(x_hbm, src_hbm, dst_hbm, ea_hbm, out_hbm,
                     accum, src_v, dst_v, ea_v, g_v,
                     src_w, dst_w, ea_w, g_w,
                     sem_g, sem_e, sem_g2, sem_e2,
                     sem_i, sem_j, sem_i2, sem_j2):
        cid = lax.axis_index("c")
        sid = lax.axis_index("s")
        wid = sid * 2 + cid

        zvec = jnp.zeros((16,), jnp.float32)

        def _zrow(r, carry):
            for k in range(D // 16):
                ea_v[r, pl.ds(k * 16, 16)] = zvec
            return carry

        lax.fori_loop(0, _CHUNK, _zrow, 0)
        zbase = sid * _RPT
        for b in range(_RPT // _CHUNK):
            pltpu.sync_copy(ea_v, accum.at[pl.ds(zbase + b * _CHUNK, _CHUNK)])
        rem = _RPT % _CHUNK
        if rem:
            pltpu.sync_copy(
                ea_v.at[pl.ds(0, rem)],
                accum.at[pl.ds(zbase + (_RPT // _CHUNK) * _CHUNK, rem)])

        @pl.when(sid == 15)
        def _():
            pltpu.sync_copy(ea_v.at[pl.ds(0, _TAIL)],
                            accum.at[pl.ds(16 * _RPT, _TAIL)])

        plsc.subcore_barrier()

        def _issue(sv, dv, ev, gv, se, sg, si, sj, j):
            chunk = j * _NW + wid

            @pl.when(chunk < _NCHUNKS)
            def _():
                base = chunk * _CHUNK
                pltpu.async_copy(src_hbm.at[pl.ds(base, _CHUNK)], sv, si)
                pltpu.async_copy(dst_hbm.at[pl.ds(base, _CHUNK)], dv, sj)
                pltpu.async_copy(ea_hbm.at[pl.ds(base, _CHUNK)], ev, se)
                pltpu.make_async_copy(
                    src_hbm.at[pl.ds(0, _CHUNK)], sv, si).wait()
                pltpu.async_copy(x_hbm.at[sv], gv, sg)

        def _finish(sv, dv, ev, gv, se, sg, si, sj, j):
            chunk = j * _NW + wid

            @pl.when(chunk < _NCHUNKS)
            def _():
                pltpu.make_async_copy(
                    dst_hbm.at[pl.ds(0, _CHUNK)], dv, sj).wait()
                pltpu.make_async_copy(
                    ea_hbm.at[pl.ds(0, _CHUNK)], ev, se).wait()
                pltpu.make_async_copy(
                    x_hbm.at[pl.ds(0, _CHUNK)], gv, sg).wait()
                pltpu.async_copy(ev, accum.at[dv], se, add=True)
                pltpu.sync_copy(gv, accum.at[dv], add=True)
                pltpu.make_async_copy(
                    ea_hbm.at[pl.ds(0, _CHUNK)], ev, se).wait()

        buf0 = (src_v, dst_v, ea_v, g_v, sem_e, sem_g, sem_i, sem_j)
        buf1 = (src_w, dst_w, ea_w, g_w, sem_e2, sem_g2, sem_i2, sem_j2)

        _issue(*buf0, 0)

        def _pair_body(t, carry):
            j0 = t * 2
            _issue(*buf1, j0 + 1)
            _finish(*buf0, j0)
            _issue(*buf0, j0 + 2)
            _finish(*buf1, j0 + 1)
            return carry

        lax.fori_loop(0, (_JMAX + 1) // 2, _pair_body, 0)
        plsc.subcore_barrier()

        obase = cid * N + sid * _RPT
        pltpu.sync_copy(accum.at[pl.ds(sid * _RPT, _RPT)],
                        out_hbm.at[pl.ds(obase, _RPT)])

        @pl.when(sid == 15)
        def _():
            pltpu.sync_copy(accum.at[pl.ds(16 * _RPT, _TAIL)],
                            out_hbm.at[pl.ds(cid * N + 16 * _RPT, _TAIL)])

    return sc_aggregate


_sc_aggregate = _make_sc_aggregate()


_BLK = 1000


def _mlp_body(p_ref, x_ref, w1_ref, b1_ref, w2_ref, b2_ref, eps_ref, o_ref):
    aggr = p_ref[0] + p_ref[1]
    h = jnp.dot(aggr, w1_ref[...], preferred_element_type=jnp.float32)
    h = jnp.maximum(h + b1_ref[...], 0.0)
    o = jnp.dot(h, w2_ref[...], preferred_element_type=jnp.float32)
    o_ref[...] = o + b2_ref[...] + (1.0 + eps_ref[0, 0]) * x_ref[...]


def _mlp(partial2, x, W1, b1, W2, b2, eps):
    grid = (N // _BLK,)
    return pl.pallas_call(
        _mlp_body,
        grid=grid,
        in_specs=[
            pl.BlockSpec((2, _BLK, D), lambda i: (0, i, 0)),
            pl.BlockSpec((_BLK, D), lambda i: (i, 0)),
            pl.BlockSpec((D, H), lambda i: (0, 0)),
            pl.BlockSpec((1, H), lambda i: (0, 0)),
            pl.BlockSpec((H, D), lambda i: (0, 0)),
            pl.BlockSpec((1, D), lambda i: (0, 0)),
            pl.BlockSpec((1, 1), lambda i: (0, 0)),
        ],
        out_specs=pl.BlockSpec((_BLK, D), lambda i: (i, 0)),
        out_shape=jax.ShapeDtypeStruct((N, D), jnp.float32),
    )(partial2, x, W1, b1, W2, b2, eps)


def kernel(x, edge_index, edge_attr, W1, b1, W2, b2, eps):
    src = edge_index[0]
    dst = edge_index[1]
    partial = _sc_aggregate(x, src, dst, edge_attr)
    partial2 = partial.reshape(2, N, D)
    return _mlp(partial2, x, W1, b1.reshape(1, H), W2, b2.reshape(1, D),
                eps.reshape(1, 1))
